# baseline (speedup 1.0000x reference)
import functools

import jax
import jax.numpy as jnp
from jax import lax
from jax.experimental import pallas as pl
from jax.experimental.pallas import tpu as pltpu
from jax.experimental.pallas import tpu_sc as plsc

ENTITY_N = 100000
RELATION_N = 1000
D = 128
B = 4096
NW = 32
BPW = B // NW
BT = 512


def _make_sc_gather():
    mesh = plsc.VectorSubcoreMesh(core_axis_name="c", subcore_axis_name="s")

    @functools.partial(
        pl.kernel,
        out_type=jax.ShapeDtypeStruct((3, B, D), jnp.float32),
        mesh=mesh,
        compiler_params=pltpu.CompilerParams(needs_layout_passes=False),
        scratch_types=[
            pltpu.VMEM((BPW, 3), jnp.int32),
            pltpu.VMEM((BPW,), jnp.int32),
            pltpu.VMEM((BPW,), jnp.int32),
            pltpu.VMEM((BPW,), jnp.int32),
            pltpu.VMEM((BPW, D), jnp.float32),
            pltpu.VMEM((BPW, D), jnp.float32),
            pltpu.VMEM((BPW, D), jnp.float32),
            pltpu.SemaphoreType.DMA,
        ],
    )
    def body(sample, entity, relation, out,
             sblk, ih_v, ir_v, it_v, buf_h, buf_r, buf_t, sem):
        wid = lax.axis_index("s") * 2 + lax.axis_index("c")
        b0 = wid * BPW
        lanes = lax.iota(jnp.int32, 16)

        pltpu.sync_copy(sample.at[pl.ds(b0, BPW)], sblk)
        for m in range(BPW // 16):
            rows = m * 16 + lanes
            for c, dst in ((0, ih_v), (1, ir_v), (2, it_v)):
                col = jnp.full((16,), c, jnp.int32)
                dst[pl.ds(m * 16, 16)] = plsc.load_gather(sblk, [rows, col])

        ch = pltpu.async_copy(entity.at[ih_v], buf_h, sem)
        cr = pltpu.async_copy(relation.at[ir_v], buf_r, sem)
        ct = pltpu.async_copy(entity.at[it_v], buf_t, sem)
        ch.wait()
        pltpu.sync_copy(buf_h, out.at[0, pl.ds(b0, BPW)])
        cr.wait()
        pltpu.sync_copy(buf_r, out.at[1, pl.ds(b0, BPW)])
        ct.wait()
        pltpu.sync_copy(buf_t, out.at[2, pl.ds(b0, BPW)])

    return body


_sc_gather = _make_sc_gather()


def _tc_norm_body(x_ref, o_ref):
    x = x_ref[0]
    ones = jnp.ones((D, D), jnp.float32)
    s = jax.lax.dot(x * x, ones, precision=jax.lax.Precision.HIGHEST)
    o_ref[0] = x / jnp.maximum(jnp.sqrt(s), 1e-12)


_tc_norm = pl.pallas_call(
    _tc_norm_body,
    grid=(3, B // BT),
    in_specs=[pl.BlockSpec((1, BT, D), lambda c, i: (c, i, 0))],
    out_specs=pl.BlockSpec((1, BT, D), lambda c, i: (c, i, 0)),
    out_shape=jax.ShapeDtypeStruct((3, B, D), jnp.float32),
)


def kernel(sample, entity_emb, relation_emb, loss_emb):
    del loss_emb
    g = _sc_gather(sample.astype(jnp.int32), entity_emb, relation_emb)
    return _tc_norm(g).transpose(1, 0, 2)
